# Initial kernel scaffold; baseline (speedup 1.0000x reference)
import jax
import jax.numpy as jnp
from jax.experimental import pallas as pl


def kernel(x, edge_index, batch, W1, b1, W2, b2, W3, b3, Wr, br, Wf, bf, Wo, bo):
    raise NotImplementedError("write your pallas kernel here")


# speedup vs baseline: 10.4843x; 10.4843x over previous
import functools

import jax
import jax.numpy as jnp
from jax import lax
from jax.experimental import pallas as pl
from jax.experimental.pallas import tpu as pltpu
from jax.experimental.pallas import tpu_sc as plsc

NC = 2
NS = 16
LANES = 16
NW = NC * NS

DEGW = 128


def _chunk_size(n):
    for c in range(128, 0, -8):
        if n % c == 0:
            return c
    raise ValueError(n)


def _npad(n, c):
    unit = NS * c
    return ((n + unit - 1) // unit) * unit


def _sc_mesh():
    return plsc.VectorSubcoreMesh(
        core_axis_name="c", subcore_axis_name="s", num_cores=NC, num_subcores=NS
    )


@functools.cache
def _sc_degree(E, N):
    EW = E // NW
    C = _chunk_size(EW)
    NCHUNK = EW // C
    NPAD = _npad(N, C)
    RPT = NPAD // NS
    KOUT = RPT // C

    @functools.partial(
        pl.kernel,
        out_type=jax.ShapeDtypeStruct((NC * NPAD, DEGW), jnp.float32),
        mesh=_sc_mesh(),
        scratch_types=[
            pltpu.VMEM((C,), jnp.int32),
            pltpu.VMEM((C, DEGW), jnp.float32),
            pltpu.VMEM((C, DEGW), jnp.float32),
            pltpu.VMEM_SHARED((NPAD, DEGW), jnp.float32),
        ],
    )
    def deg_kernel(dst_hbm, ones_hbm, out_hbm, dst_v, rows_v, zbuf_v, acc_sh):
        c = lax.axis_index("c")
        s = lax.axis_index("s")
        wid = c * NS + s
        row0 = s * RPT

        pltpu.sync_copy(ones_hbm, rows_v)

        def zrow(r, carry):
            for j in range(DEGW // LANES):
                zbuf_v[r, pl.ds(j * LANES, LANES)] = jnp.zeros((LANES,), jnp.float32)
            return carry

        lax.fori_loop(0, C, zrow, 0)

        def zstripe(k, carry):
            pltpu.sync_copy(zbuf_v, acc_sh.at[pl.ds(row0 + k * C, C)])
            return carry

        lax.fori_loop(0, KOUT, zstripe, 0)

        plsc.subcore_barrier()

        ebase = wid * EW

        def chunk(i, carry):
            b = ebase + i * C
            pltpu.sync_copy(dst_hbm.at[pl.ds(b, C)], dst_v)
            pltpu.sync_copy(rows_v, acc_sh.at[dst_v], add=True)
            return carry

        lax.fori_loop(0, NCHUNK, chunk, 0)

        plsc.subcore_barrier()

        obase = c * NPAD + row0

        def wout(k, carry):
            pltpu.sync_copy(acc_sh.at[pl.ds(row0 + k * C, C)], zbuf_v)
            pltpu.sync_copy(zbuf_v, out_hbm.at[pl.ds(obase + k * C, C)])
            return carry

        lax.fori_loop(0, KOUT, wout, 0)

    return deg_kernel


@functools.cache
def _sc_row_scatter(E, N, D):
    EW = E // NW
    C = _chunk_size(EW)
    NCHUNK = EW // C
    NPAD = _npad(N, C)
    RPT = NPAD // NS
    KOUT = RPT // C

    @functools.partial(
        pl.kernel,
        out_type=jax.ShapeDtypeStruct((NC * NPAD, D), jnp.float32),
        mesh=_sc_mesh(),
        scratch_types=[
            pltpu.VMEM((C,), jnp.int32),
            pltpu.VMEM((C,), jnp.int32),
            pltpu.VMEM((C, D), jnp.float32),
            pltpu.SemaphoreType.DMA,
            pltpu.VMEM_SHARED((NPAD, D), jnp.float32),
        ],
    )
    def scat_kernel(hs_hbm, src_hbm, dst_hbm, out_hbm, src_v, dst_v, rows_v, sem, acc_sh):
        c = lax.axis_index("c")
        s = lax.axis_index("s")
        wid = c * NS + s
        row0 = s * RPT

        def zrow(r, carry):
            for j in range(D // LANES):
                rows_v[r, pl.ds(j * LANES, LANES)] = jnp.zeros((LANES,), jnp.float32)
            return carry

        lax.fori_loop(0, C, zrow, 0)

        def zstripe(k, carry):
            pltpu.sync_copy(rows_v, acc_sh.at[pl.ds(row0 + k * C, C)])
            return carry

        lax.fori_loop(0, KOUT, zstripe, 0)

        plsc.subcore_barrier()

        ebase = wid * EW

        def chunk(i, carry):
            b = ebase + i * C
            pltpu.sync_copy(src_hbm.at[pl.ds(b, C)], src_v)
            pltpu.sync_copy(dst_hbm.at[pl.ds(b, C)], dst_v)
            pltpu.async_copy(hs_hbm.at[src_v], rows_v, sem).wait()
            pltpu.sync_copy(rows_v, acc_sh.at[dst_v], add=True)
            return carry

        lax.fori_loop(0, NCHUNK, chunk, 0)

        plsc.subcore_barrier()

        obase = c * NPAD + row0

        def wout(k, carry):
            pltpu.sync_copy(acc_sh.at[pl.ds(row0 + k * C, C)], rows_v)
            pltpu.sync_copy(rows_v, out_hbm.at[pl.ds(obase + k * C, C)])
            return carry

        lax.fori_loop(0, KOUT, wout, 0)

    return scat_kernel


_BR = 400


def _dinv_block(d0_ref, d1_ref):
    cnt = d0_ref[0, :, 0:1] + d1_ref[0, :, 0:1]
    return lax.rsqrt(cnt + 1.0)


@functools.cache
def _tc_pre(N, Din, Dout, NPAD):
    G = N // _BR

    def body(x_ref, w_ref, d0_ref, d1_ref, hs_ref):
        dinv = _dinv_block(d0_ref, d1_ref)
        h = jnp.dot(x_ref[...], w_ref[...], preferred_element_type=jnp.float32)
        hs_ref[...] = h * dinv

    return pl.pallas_call(
        body,
        grid=(G,),
        in_specs=[
            pl.BlockSpec((_BR, Din), lambda i: (i, 0)),
            pl.BlockSpec((Din, Dout), lambda i: (0, 0)),
            pl.BlockSpec((1, _BR, DEGW), lambda i: (0, i, 0)),
            pl.BlockSpec((1, _BR, DEGW), lambda i: (1, i, 0)),
        ],
        out_specs=pl.BlockSpec((_BR, Dout), lambda i: (i, 0)),
        out_shape=jax.ShapeDtypeStruct((N, Dout), jnp.float32),
    )


@functools.cache
def _tc_mid(N, D, Dout, NPAD):
    G = N // _BR

    def body(a0_ref, a1_ref, hs_ref, d0_ref, d1_ref, b_ref, w_ref, h_ref, hs2_ref):
        dinv = _dinv_block(d0_ref, d1_ref)
        h = dinv * (a0_ref[0] + a1_ref[0] + hs_ref[...]) + b_ref[...]
        h_ref[...] = h
        hs2_ref[...] = jnp.dot(h, w_ref[...], preferred_element_type=jnp.float32) * dinv

    return pl.pallas_call(
        body,
        grid=(G,),
        in_specs=[
            pl.BlockSpec((1, _BR, D), lambda i: (0, i, 0)),
            pl.BlockSpec((1, _BR, D), lambda i: (1, i, 0)),
            pl.BlockSpec((_BR, D), lambda i: (i, 0)),
            pl.BlockSpec((1, _BR, DEGW), lambda i: (0, i, 0)),
            pl.BlockSpec((1, _BR, DEGW), lambda i: (1, i, 0)),
            pl.BlockSpec((1, D), lambda i: (0, 0)),
            pl.BlockSpec((D, Dout), lambda i: (0, 0)),
        ],
        out_specs=[
            pl.BlockSpec((_BR, D), lambda i: (i, 0)),
            pl.BlockSpec((_BR, Dout), lambda i: (i, 0)),
        ],
        out_shape=[
            jax.ShapeDtypeStruct((N, D), jnp.float32),
            jax.ShapeDtypeStruct((N, Dout), jnp.float32),
        ],
    )


@functools.cache
def _tc_final(N, D, NPAD, NG, DR, DF, DO):
    G = N // _BR

    def body(a0_ref, a1_ref, hs_ref, d0_ref, d1_ref, b_ref, h1_ref, h2_ref,
             bat_ref, wr_ref, br_ref, wf_ref, bf_ref, wo_ref, bo_ref,
             out_ref, p1, p2, p3, pcnt):
        i = pl.program_id(0)
        dinv = _dinv_block(d0_ref, d1_ref)
        h3 = dinv * (a0_ref[0] + a1_ref[0] + hs_ref[...]) + b_ref[...]

        gi = lax.broadcasted_iota(jnp.int32, (NG, _BR), 0)
        oh = (gi == bat_ref[0]).astype(jnp.float32)

        @pl.when(i == 0)
        def _():
            p1[...] = jnp.zeros_like(p1)
            p2[...] = jnp.zeros_like(p2)
            p3[...] = jnp.zeros_like(p3)
            pcnt[...] = jnp.zeros_like(pcnt)

        p1[...] += jnp.dot(oh, h1_ref[...], preferred_element_type=jnp.float32)
        p2[...] += jnp.dot(oh, h2_ref[...], preferred_element_type=jnp.float32)
        p3[...] += jnp.dot(oh, h3, preferred_element_type=jnp.float32)
        pcnt[...] += jnp.sum(oh, axis=1, keepdims=True)

        @pl.when(i == G - 1)
        def _():
            inv = 1.0 / jnp.maximum(pcnt[:, 0:1], 1.0)
            q1 = p1[...] * inv
            q2 = p2[...] * inv
            q3 = p3[...] * inv
            r = (jnp.dot(q1, wr_ref[0:D, :], preferred_element_type=jnp.float32)
                 + jnp.dot(q2, wr_ref[D:2 * D, :], preferred_element_type=jnp.float32)
                 + jnp.dot(q3, wr_ref[2 * D:3 * D, :], preferred_element_type=jnp.float32)
                 + br_ref[...])
            r = jnp.maximum(r, 0.0)
            f = jnp.maximum(jnp.dot(r, wf_ref[...], preferred_element_type=jnp.float32)
                            + bf_ref[...], 0.0)
            out_ref[...] = jnp.dot(f, wo_ref[...], preferred_element_type=jnp.float32) \
                + bo_ref[...]

    return pl.pallas_call(
        body,
        grid=(G,),
        in_specs=[
            pl.BlockSpec((1, _BR, D), lambda i: (0, i, 0)),
            pl.BlockSpec((1, _BR, D), lambda i: (1, i, 0)),
            pl.BlockSpec((_BR, D), lambda i: (i, 0)),
            pl.BlockSpec((1, _BR, DEGW), lambda i: (0, i, 0)),
            pl.BlockSpec((1, _BR, DEGW), lambda i: (1, i, 0)),
            pl.BlockSpec((1, D), lambda i: (0, 0)),
            pl.BlockSpec((_BR, D), lambda i: (i, 0)),
            pl.BlockSpec((_BR, D), lambda i: (i, 0)),
            pl.BlockSpec((1, 1, _BR), lambda i: (i, 0, 0)),
            pl.BlockSpec((3 * D, DR), lambda i: (0, 0)),
            pl.BlockSpec((1, DR), lambda i: (0, 0)),
            pl.BlockSpec((DR, DF), lambda i: (0, 0)),
            pl.BlockSpec((1, DF), lambda i: (0, 0)),
            pl.BlockSpec((DF, DO), lambda i: (0, 0)),
            pl.BlockSpec((1, DO), lambda i: (0, 0)),
        ],
        out_specs=pl.BlockSpec((NG, DO), lambda i: (0, 0)),
        out_shape=jax.ShapeDtypeStruct((NG, DO), jnp.float32),
        scratch_shapes=[
            pltpu.VMEM((NG, D), jnp.float32),
            pltpu.VMEM((NG, D), jnp.float32),
            pltpu.VMEM((NG, D), jnp.float32),
            pltpu.VMEM((NG, 128), jnp.float32),
        ],
    )


def kernel(x, edge_index, batch, W1, b1, W2, b2, W3, b3, Wr, br, Wf, bf, Wo, bo):
    N, Din = x.shape
    E = edge_index.shape[1]
    D = W1.shape[1]
    NG = 16
    DR = Wr.shape[1]
    DF = Wf.shape[1]
    DO = Wo.shape[1]
    C = _chunk_size(E // NW)
    NPAD = _npad(N, C)

    src = edge_index[0]
    dst = edge_index[1]

    ones = jnp.ones((C, DEGW), jnp.float32)
    dega = _sc_degree(E, N)(dst, ones).reshape(NC, NPAD, DEGW)

    hs1 = _tc_pre(N, Din, D, NPAD)(x, W1, dega, dega)

    scat = _sc_row_scatter(E, N, D)
    acc1 = scat(hs1, src, dst).reshape(NC, NPAD, D)

    mid = _tc_mid(N, D, D, NPAD)
    h1, hs2 = mid(acc1, acc1, hs1, dega, dega, b1.reshape(1, D), W2)

    acc2 = scat(hs2, src, dst).reshape(NC, NPAD, D)
    h2, hs3 = mid(acc2, acc2, hs2, dega, dega, b2.reshape(1, D), W3)

    acc3 = scat(hs3, src, dst).reshape(NC, NPAD, D)

    bat3 = batch.reshape(N // _BR, 1, _BR)
    out = _tc_final(N, D, NPAD, NG, DR, DF, DO)(
        acc3, acc3, hs3, dega, dega, b3.reshape(1, D), h1, h2,
        bat3, Wr, br.reshape(1, DR), Wf, bf.reshape(1, DF), Wo, bo.reshape(1, DO))
    return out
